# baseline (speedup 1.0000x reference)
import jax
import jax.numpy as jnp
from jax import lax
from jax.experimental import pallas as pl
from jax.experimental.pallas import tpu as pltpu
from jax.experimental.pallas import tpu_sc as plsc

NMAX = 48
D = 64
L = 3
PAIR = NMAX * NMAX
UB = 8
NU = NMAX // UB

N = 4096
E = 16384
B = 128
NS = 16
NC = 2
EPT = E // NS
NPT = N // NS
GPP = 4
SLAB = GPP * PAIR
ROWS_PT = SLAB // NS
PASSES = (B // NC) // GPP

_INTERPRET = False


def _sc_body(at_hbm, bt_hbm, xi_hbm, ei_hbm, fi_hbm, nfi_hbm, zz_hbm, a_out,
             ev_v, hv_v, st_v, xi_v, ei_v, fi_v, nfi_v,
             idx_v, idxn_v, slab, sem):
    f32 = jnp.float32
    c = lax.axis_index("c")
    s = lax.axis_index("s")

    descs = []
    for f in range(9):
        descs.append(pltpu.async_copy(xi_hbm.at[f, pl.ds(s * 2, 2)],
                                      xi_v.at[pl.ds(f * 2, 2)], sem))
    for f in range(3):
        descs.append(pltpu.async_copy(ei_hbm.at[f, pl.ds(s * 8, 8)],
                                      ei_v.at[pl.ds(f * 8, 8)], sem))
    descs.append(pltpu.async_copy(fi_hbm.at[pl.ds(s * 8, 8)], fi_v, sem))
    descs.append(pltpu.async_copy(nfi_hbm.at[pl.ds(s * 2, 2)], nfi_v, sem))
    for dd in descs:
        dd.wait()

    descs = [pltpu.async_copy(bt_hbm.at[ei_v.at[k]],
                              ev_v.at[pl.ds(k * 128, 128)], sem)
             for k in range(8)]
    for dd in descs:
        dd.wait()
    for f in (1, 2):
        last = f == 2
        for j in range(16):
            pltpu.sync_copy(
                bt_hbm.at[ei_v.at[f * 8 + j // 2, pl.ds((j % 2) * 64, 64)]],
                st_v)

            def eadd(i, carry, _j=j, _last=last):
                for jj in range(4):
                    v = (ev_v[_j * 64 + i, pl.ds(jj * 16, 16)]
                         + st_v[i, pl.ds(jj * 16, 16)])
                    if _last:
                        v = jnp.maximum(v, 0.0)
                    ev_v[_j * 64 + i, pl.ds(jj * 16, 16)] = v
                return carry

            lax.fori_loop(0, 64, eadd, 0)

    descs = [pltpu.async_copy(at_hbm.at[xi_v.at[k]],
                              hv_v.at[pl.ds(k * 128, 128)], sem)
             for k in range(2)]
    for dd in descs:
        dd.wait()
    for f in range(1, 9):
        last = f == 8
        for j in range(4):
            pltpu.sync_copy(
                at_hbm.at[xi_v.at[f * 2 + j // 2, pl.ds((j % 2) * 64, 64)]],
                st_v)

            def hadd(i, carry, _j=j, _last=last):
                for jj in range(4):
                    v = (hv_v[_j * 64 + i, pl.ds(jj * 16, 16)]
                         + st_v[i, pl.ds(jj * 16, 16)])
                    if _last:
                        v = jnp.maximum(v, 0.0)
                    hv_v[_j * 64 + i, pl.ds(jj * 16, 16)] = v
                return carry

            lax.fori_loop(0, 64, hadd, 0)

    trash = jnp.int32(SLAB) + s
    for p in range(PASSES):
        base = (c * (B // NC) + p * GPP) * PAIR
        d0 = pltpu.async_copy(
            zz_hbm.at[pl.ds(s * ROWS_PT, ROWS_PT)],
            slab.at[pl.ds(s * ROWS_PT, ROWS_PT)], sem)
        d1 = pltpu.async_copy(
            zz_hbm.at[pl.ds(SLAB + s, 1)], slab.at[pl.ds(SLAB + s, 1)], sem)
        d0.wait()
        d1.wait()
        plsc.subcore_barrier()

        def eadj(j, carry):
            for k in range(8):
                t = fi_v[k, pl.ds(j * 16, 16)] - base
                ok = (t >= 0) & (t < SLAB)
                idx_v[k, pl.ds(j * 16, 16)] = jnp.where(ok, t, trash)
            return carry

        lax.fori_loop(0, 8, eadj, 0)

        def nadj(j, carry):
            for k in range(2):
                t = nfi_v[k, pl.ds(j * 16, 16)] - base
                ok = (t >= 0) & (t < SLAB)
                idxn_v[k, pl.ds(j * 16, 16)] = jnp.where(ok, t, trash)
            return carry

        lax.fori_loop(0, 8, nadj, 0)

        for k in range(8):
            def ecnt(j, acc, _k=k):
                grp = idx_v[_k, pl.ds(j * 16, 16)]
                return acc + jnp.where(grp != trash, 1, 0)

            tot = lax.fori_loop(
                0, 8, ecnt, jnp.zeros((16,), jnp.int32)).sum()

            @pl.when(tot > 0)
            def _(_k=k):
                pltpu.sync_copy(ev_v.at[pl.ds(_k * 128, 128)],
                                slab.at[idx_v.at[_k]], add=True)
        for k in range(2):
            def ncnt(j, acc, _k=k):
                grp = idxn_v[_k, pl.ds(j * 16, 16)]
                return acc + jnp.where(grp != trash, 1, 0)

            tot = lax.fori_loop(
                0, 8, ncnt, jnp.zeros((16,), jnp.int32)).sum()

            @pl.when(tot > 0)
            def _(_k=k):
                pltpu.sync_copy(hv_v.at[pl.ds(_k * 128, 128)],
                                slab.at[idxn_v.at[_k]], add=True)
        plsc.subcore_barrier()

        pltpu.sync_copy(slab.at[pl.ds(s * ROWS_PT, ROWS_PT)],
                        a_out.at[pl.ds(base + s * ROWS_PT, ROWS_PT)])
        plsc.subcore_barrier()


def _sc_frontend(at_flat, bt_flat, xi3, ei3, fi2, nfi2, zz):
    f32 = jnp.float32
    i32 = jnp.int32
    mesh = plsc.VectorSubcoreMesh(core_axis_name="c", subcore_axis_name="s")
    fn = pl.kernel(
        _sc_body,
        out_type=jax.ShapeDtypeStruct((B * PAIR, D), f32),
        mesh=mesh,
        scratch_types=[
            pltpu.VMEM((EPT, D), f32),
            pltpu.VMEM((NPT, D), f32),
            pltpu.VMEM((64, D), f32),
            pltpu.VMEM((18, 128), i32),
            pltpu.VMEM((24, 128), i32),
            pltpu.VMEM((8, 128), i32),
            pltpu.VMEM((2, 128), i32),
            pltpu.VMEM((8, 128), i32),
            pltpu.VMEM((2, 128), i32),
            pltpu.VMEM_SHARED((SLAB + NS, D), f32),
            pltpu.SemaphoreType.DMA,
        ],
        compiler_params=pltpu.CompilerParams(use_tc_tiling_on_sc=False,
                                             needs_layout_passes=False),
    )
    return fn(at_flat, bt_flat, xi3, ei3, fi2, nfi2, zz)


def _tc_kernel(cnt_ref, a_ref, w1_ref, w2_ref, w3_ref,
               wout_ref, bout_ref, out_ref, h1t_scr, h2_scr, x_scr):
    b = pl.program_id(0)
    nv = jnp.minimum(cnt_ref[b], NMAX)
    f32 = jnp.float32

    r = lax.broadcasted_iota(jnp.int32, (PAIR, 1), 0)
    pmf = ((r // NMAX < nv) & (r % NMAX < nv)).astype(f32)
    cntp = (nv * nv).astype(f32) + 1e-6

    Z = a_ref[...]
    Zt = jnp.swapaxes(Z.reshape(NMAX, NMAX, D), 0, 1).reshape(PAIR, D)

    for l in range(L):
        h1t_scr[...] = jnp.maximum(
            jnp.dot(Zt, w1_ref[l], preferred_element_type=f32), 0.0)
        h2_scr[...] = jnp.maximum(
            jnp.dot(Z, w2_ref[l], preferred_element_type=f32), 0.0)
        zw3 = jnp.dot(Z, w3_ref[l], preferred_element_type=f32)
        x_scr[...] = zw3

        nblk = (nv + 7) // 8
        for ub in range(NU):
            def ein_body(wb, acc, _ub=ub):
                base = wb * (8 * NMAX)
                for j in range(8):
                    a = h1t_scr[pl.ds(base + j * NMAX + _ub * UB, UB), :]
                    bb = h2_scr[pl.ds(base + j * NMAX, NMAX), :]
                    acc = acc + a[:, None, :] * bb[None, :, :]
                return acc

            acc = lax.fori_loop(
                0, nblk, ein_body, jnp.zeros((UB, NMAX, D), f32))
            x_scr[pl.ds(ub * UB * NMAX, UB * NMAX), :] += acc.reshape(
                UB * NMAX, D)

        X = x_scr[...]
        mu = X.sum(axis=0) / cntp
        x2 = (X * X).sum(axis=0) / cntp
        var = x2 - mu * mu
        s = lax.rsqrt(var + 1e-5)
        Znew = jnp.maximum((X - mu[None, :]) * s[None, :], 0.0) * pmf
        Z = Znew
        if l < L - 1:
            Zt = jnp.swapaxes(
                Znew.reshape(NMAX, NMAX, D), 0, 1).reshape(PAIR, D)

    g = Z.sum(axis=0) / cntp
    val = (g * wout_ref[0, :]).sum() + bout_ref[0]
    out_ref[0, 0, :] = jnp.full((128,), val, dtype=jnp.float32)


def _dense_backend(A, cnt, W1, W2, W3, Wout, bout):
    out = pl.pallas_call(
        _tc_kernel,
        grid=(B,),
        in_specs=[
            pl.BlockSpec(memory_space=pltpu.SMEM),
            pl.BlockSpec((PAIR, D), lambda b: (b, 0)),
            pl.BlockSpec(W1.shape, lambda b: (0, 0, 0)),
            pl.BlockSpec(W2.shape, lambda b: (0, 0, 0)),
            pl.BlockSpec(W3.shape, lambda b: (0, 0, 0)),
            pl.BlockSpec((1, D), lambda b: (0, 0)),
            pl.BlockSpec(memory_space=pltpu.SMEM),
        ],
        out_specs=pl.BlockSpec((1, 1, 128), lambda b: (b, 0, 0)),
        out_shape=jax.ShapeDtypeStruct((B, 1, 128), jnp.float32),
        scratch_shapes=[pltpu.VMEM((PAIR, D), jnp.float32),
                        pltpu.VMEM((PAIR, D), jnp.float32),
                        pltpu.VMEM((PAIR, D), jnp.float32)],
        compiler_params=pltpu.CompilerParams(
            dimension_semantics=("arbitrary",)),
        interpret=_INTERPRET,
    )(cnt, A, W1, W2, W3, Wout.T, bout)
    return out[:, 0, :1]


def kernel(x, edge_index, edge_attr, batch0, atom_tables, bond_tables,
           W1, W2, W3, Wout, bout):
    i32 = jnp.int32

    counts = jnp.bincount(batch0, length=B)
    offsets = jnp.cumsum(counts) - counts
    local = jnp.arange(N, dtype=i32) - offsets[batch0].astype(i32)
    nvalid = local < NMAX
    lc = jnp.minimum(local, NMAX - 1)

    src, dst = edge_index[0], edge_index[1]
    gs = batch0[src]
    gd = batch0[dst]
    ls = lc[src]
    ld = lc[dst]
    ev = (gs == gd) & nvalid[src] & nvalid[dst]
    fi = jnp.where(ev, gs.astype(i32) * PAIR + ls * NMAX + ld, B * PAIR)
    nfi = jnp.where(nvalid, batch0.astype(i32) * PAIR + lc * (NMAX + 1),
                    B * PAIR)

    xi3 = (x.astype(i32) + jnp.arange(9, dtype=i32)[None, :] * 64
           ).T.reshape(9, N // 128, 128)
    ei3 = (edge_attr.astype(i32) + jnp.arange(3, dtype=i32)[None, :] * 4
           ).T.reshape(3, E // 128, 128)
    fi2 = fi.reshape(E // 128, 128)
    nfi2 = nfi.reshape(N // 128, 128)
    at_flat = atom_tables.reshape(9 * 64, D)
    bt_flat = bond_tables.reshape(3 * 4, D)

    zz = jnp.zeros((SLAB + NS, D), jnp.float32)
    A = _sc_frontend(at_flat, bt_flat, xi3, ei3, fi2, nfi2, zz)

    return _dense_backend(A, counts.astype(i32), W1, W2, W3, Wout, bout)


# speedup vs baseline: 1.0025x; 1.0025x over previous
import jax
import jax.numpy as jnp
from jax import lax
from jax.experimental import pallas as pl
from jax.experimental.pallas import tpu as pltpu
from jax.experimental.pallas import tpu_sc as plsc

NMAX = 48
D = 64
L = 3
PAIR = NMAX * NMAX
UB = 8
NU = NMAX // UB

N = 4096
E = 16384
B = 128
NS = 16
NC = 2
EPT = E // NS
NPT = N // NS
GPP = 4
SLAB = GPP * PAIR
ROWS_PT = SLAB // NS
PASSES = (B // NC) // GPP

_INTERPRET = False


def _sc_body(at_hbm, bt_hbm, xi_hbm, ei_hbm, fi_hbm, nfi_hbm, zz_hbm, a_out,
             ev_v, hv_v, st_v, xi_v, ei_v, fi_v, nfi_v,
             idx_v, idxn_v, slab, sem):
    f32 = jnp.float32
    c = lax.axis_index("c")
    s = lax.axis_index("s")

    descs = []
    for f in range(9):
        descs.append(pltpu.async_copy(xi_hbm.at[f, pl.ds(s * 2, 2)],
                                      xi_v.at[pl.ds(f * 2, 2)], sem))
    for f in range(3):
        descs.append(pltpu.async_copy(ei_hbm.at[f, pl.ds(s * 8, 8)],
                                      ei_v.at[pl.ds(f * 8, 8)], sem))
    descs.append(pltpu.async_copy(fi_hbm.at[pl.ds(s * 8, 8)], fi_v, sem))
    descs.append(pltpu.async_copy(nfi_hbm.at[pl.ds(s * 2, 2)], nfi_v, sem))
    for dd in descs:
        dd.wait()

    scope_encode = jax.named_scope("sc_encode")
    scope_encode.__enter__()
    descs = [pltpu.async_copy(bt_hbm.at[ei_v.at[k]],
                              ev_v.at[pl.ds(k * 128, 128)], sem)
             for k in range(8)]
    for dd in descs:
        dd.wait()
    for f in (1, 2):
        last = f == 2
        for j in range(16):
            pltpu.sync_copy(
                bt_hbm.at[ei_v.at[f * 8 + j // 2, pl.ds((j % 2) * 64, 64)]],
                st_v)

            def eadd(i, carry, _j=j, _last=last):
                for jj in range(4):
                    v = (ev_v[_j * 64 + i, pl.ds(jj * 16, 16)]
                         + st_v[i, pl.ds(jj * 16, 16)])
                    if _last:
                        v = jnp.maximum(v, 0.0)
                    ev_v[_j * 64 + i, pl.ds(jj * 16, 16)] = v
                return carry

            lax.fori_loop(0, 64, eadd, 0)

    descs = [pltpu.async_copy(at_hbm.at[xi_v.at[k]],
                              hv_v.at[pl.ds(k * 128, 128)], sem)
             for k in range(2)]
    for dd in descs:
        dd.wait()
    for f in range(1, 9):
        last = f == 8
        for j in range(4):
            pltpu.sync_copy(
                at_hbm.at[xi_v.at[f * 2 + j // 2, pl.ds((j % 2) * 64, 64)]],
                st_v)

            def hadd(i, carry, _j=j, _last=last):
                for jj in range(4):
                    v = (hv_v[_j * 64 + i, pl.ds(jj * 16, 16)]
                         + st_v[i, pl.ds(jj * 16, 16)])
                    if _last:
                        v = jnp.maximum(v, 0.0)
                    hv_v[_j * 64 + i, pl.ds(jj * 16, 16)] = v
                return carry

            lax.fori_loop(0, 64, hadd, 0)

    scope_encode.__exit__(None, None, None)

    trash = jnp.int32(SLAB) + s
    for p in range(PASSES):
        base = (c * (B // NC) + p * GPP) * PAIR
        with jax.named_scope("sc_zero"):
            d0 = pltpu.async_copy(
                zz_hbm.at[pl.ds(s * ROWS_PT, ROWS_PT)],
                slab.at[pl.ds(s * ROWS_PT, ROWS_PT)], sem)
            d1 = pltpu.async_copy(
                zz_hbm.at[pl.ds(SLAB + s, 1)],
                slab.at[pl.ds(SLAB + s, 1)], sem)
            d0.wait()
            d1.wait()
            plsc.subcore_barrier()

        scope_adj = jax.named_scope("sc_adjust")
        scope_adj.__enter__()

        def eadj(j, carry):
            for k in range(8):
                t = fi_v[k, pl.ds(j * 16, 16)] - base
                ok = (t >= 0) & (t < SLAB)
                idx_v[k, pl.ds(j * 16, 16)] = jnp.where(ok, t, trash)
            return carry

        lax.fori_loop(0, 8, eadj, 0)

        def nadj(j, carry):
            for k in range(2):
                t = nfi_v[k, pl.ds(j * 16, 16)] - base
                ok = (t >= 0) & (t < SLAB)
                idxn_v[k, pl.ds(j * 16, 16)] = jnp.where(ok, t, trash)
            return carry

        lax.fori_loop(0, 8, nadj, 0)
        scope_adj.__exit__(None, None, None)
        scope_sc = jax.named_scope("sc_scatter")
        scope_sc.__enter__()

        for k in range(8):
            def ecnt(j, acc, _k=k):
                grp = idx_v[_k, pl.ds(j * 16, 16)]
                return acc + jnp.where(grp != trash, 1, 0)

            tot = lax.fori_loop(
                0, 8, ecnt, jnp.zeros((16,), jnp.int32)).sum()

            @pl.when(tot > 0)
            def _(_k=k):
                pltpu.sync_copy(ev_v.at[pl.ds(_k * 128, 128)],
                                slab.at[idx_v.at[_k]], add=True)
        for k in range(2):
            def ncnt(j, acc, _k=k):
                grp = idxn_v[_k, pl.ds(j * 16, 16)]
                return acc + jnp.where(grp != trash, 1, 0)

            tot = lax.fori_loop(
                0, 8, ncnt, jnp.zeros((16,), jnp.int32)).sum()

            @pl.when(tot > 0)
            def _(_k=k):
                pltpu.sync_copy(hv_v.at[pl.ds(_k * 128, 128)],
                                slab.at[idxn_v.at[_k]], add=True)
        plsc.subcore_barrier()
        scope_sc.__exit__(None, None, None)

        pltpu.sync_copy(slab.at[pl.ds(s * ROWS_PT, ROWS_PT)],
                        a_out.at[pl.ds(base + s * ROWS_PT, ROWS_PT)])
        plsc.subcore_barrier()


def _sc_frontend(at_flat, bt_flat, xi3, ei3, fi2, nfi2, zz):
    f32 = jnp.float32
    i32 = jnp.int32
    mesh = plsc.VectorSubcoreMesh(core_axis_name="c", subcore_axis_name="s")
    fn = pl.kernel(
        _sc_body,
        out_type=jax.ShapeDtypeStruct((B * PAIR, D), f32),
        mesh=mesh,
        scratch_types=[
            pltpu.VMEM((EPT, D), f32),
            pltpu.VMEM((NPT, D), f32),
            pltpu.VMEM((64, D), f32),
            pltpu.VMEM((18, 128), i32),
            pltpu.VMEM((24, 128), i32),
            pltpu.VMEM((8, 128), i32),
            pltpu.VMEM((2, 128), i32),
            pltpu.VMEM((8, 128), i32),
            pltpu.VMEM((2, 128), i32),
            pltpu.VMEM_SHARED((SLAB + NS, D), f32),
            pltpu.SemaphoreType.DMA,
        ],
        compiler_params=pltpu.CompilerParams(use_tc_tiling_on_sc=False,
                                             needs_layout_passes=False),
    )
    return fn(at_flat, bt_flat, xi3, ei3, fi2, nfi2, zz)


def _tc_kernel(cnt_ref, a_ref, w1_ref, w2_ref, w3_ref,
               wout_ref, bout_ref, out_ref, h1t_scr, h2_scr, x_scr):
    b = pl.program_id(0)
    nv = jnp.minimum(cnt_ref[b], NMAX)
    f32 = jnp.float32

    r = lax.broadcasted_iota(jnp.int32, (PAIR, 1), 0)
    pmf = ((r // NMAX < nv) & (r % NMAX < nv)).astype(f32)
    cntp = (nv * nv).astype(f32) + 1e-6

    Z = a_ref[...]
    Zt = jnp.swapaxes(Z.reshape(NMAX, NMAX, D), 0, 1).reshape(PAIR, D)

    for l in range(L):
        h1t_scr[...] = jnp.maximum(
            jnp.dot(Zt, w1_ref[l], preferred_element_type=f32), 0.0)
        h2_scr[...] = jnp.maximum(
            jnp.dot(Z, w2_ref[l], preferred_element_type=f32), 0.0)
        zw3 = jnp.dot(Z, w3_ref[l], preferred_element_type=f32)
        x_scr[...] = zw3

        nblk = (nv + 7) // 8
        for ub in range(NU):
            def ein_body(wb, acc, _ub=ub):
                base = wb * (8 * NMAX)
                for j in range(8):
                    a = h1t_scr[pl.ds(base + j * NMAX + _ub * UB, UB), :]
                    bb = h2_scr[pl.ds(base + j * NMAX, NMAX), :]
                    acc = acc + a[:, None, :] * bb[None, :, :]
                return acc

            acc = lax.fori_loop(
                0, nblk, ein_body, jnp.zeros((UB, NMAX, D), f32))
            x_scr[pl.ds(ub * UB * NMAX, UB * NMAX), :] += acc.reshape(
                UB * NMAX, D)

        X = x_scr[...]
        mu = X.sum(axis=0) / cntp
        x2 = (X * X).sum(axis=0) / cntp
        var = x2 - mu * mu
        s = lax.rsqrt(var + 1e-5)
        Znew = jnp.maximum((X - mu[None, :]) * s[None, :], 0.0) * pmf
        Z = Znew
        if l < L - 1:
            Zt = jnp.swapaxes(
                Znew.reshape(NMAX, NMAX, D), 0, 1).reshape(PAIR, D)

    g = Z.sum(axis=0) / cntp
    val = (g * wout_ref[0, :]).sum() + bout_ref[0]
    out_ref[0, 0, :] = jnp.full((128,), val, dtype=jnp.float32)


def _dense_backend(A, cnt, W1, W2, W3, Wout, bout):
    out = pl.pallas_call(
        _tc_kernel,
        grid=(B,),
        in_specs=[
            pl.BlockSpec(memory_space=pltpu.SMEM),
            pl.BlockSpec((PAIR, D), lambda b: (b, 0)),
            pl.BlockSpec(W1.shape, lambda b: (0, 0, 0)),
            pl.BlockSpec(W2.shape, lambda b: (0, 0, 0)),
            pl.BlockSpec(W3.shape, lambda b: (0, 0, 0)),
            pl.BlockSpec((1, D), lambda b: (0, 0)),
            pl.BlockSpec(memory_space=pltpu.SMEM),
        ],
        out_specs=pl.BlockSpec((1, 1, 128), lambda b: (b, 0, 0)),
        out_shape=jax.ShapeDtypeStruct((B, 1, 128), jnp.float32),
        scratch_shapes=[pltpu.VMEM((PAIR, D), jnp.float32),
                        pltpu.VMEM((PAIR, D), jnp.float32),
                        pltpu.VMEM((PAIR, D), jnp.float32)],
        compiler_params=pltpu.CompilerParams(
            dimension_semantics=("arbitrary",)),
        interpret=_INTERPRET,
    )(cnt, A, W1, W2, W3, Wout.T, bout)
    return out[:, 0, :1]


def kernel(x, edge_index, edge_attr, batch0, atom_tables, bond_tables,
           W1, W2, W3, Wout, bout):
    i32 = jnp.int32

    counts = jnp.bincount(batch0, length=B)
    offsets = jnp.cumsum(counts) - counts
    local = jnp.arange(N, dtype=i32) - offsets[batch0].astype(i32)
    nvalid = local < NMAX
    lc = jnp.minimum(local, NMAX - 1)

    src, dst = edge_index[0], edge_index[1]
    gs = batch0[src]
    gd = batch0[dst]
    ls = lc[src]
    ld = lc[dst]
    ev = (gs == gd) & nvalid[src] & nvalid[dst]
    fi = jnp.where(ev, gs.astype(i32) * PAIR + ls * NMAX + ld, B * PAIR)
    nfi = jnp.where(nvalid, batch0.astype(i32) * PAIR + lc * (NMAX + 1),
                    B * PAIR)

    xi3 = (x.astype(i32) + jnp.arange(9, dtype=i32)[None, :] * 64
           ).T.reshape(9, N // 128, 128)
    ei3 = (edge_attr.astype(i32) + jnp.arange(3, dtype=i32)[None, :] * 4
           ).T.reshape(3, E // 128, 128)
    fi2 = fi.reshape(E // 128, 128)
    nfi2 = nfi.reshape(N // 128, 128)
    at_flat = atom_tables.reshape(9 * 64, D)
    bt_flat = bond_tables.reshape(3 * 4, D)

    zz = jnp.zeros((SLAB + NS, D), jnp.float32)
    A = _sc_frontend(at_flat, bt_flat, xi3, ei3, fi2, nfi2, zz)

    return _dense_backend(A, counts.astype(i32), W1, W2, W3, Wout, bout)


# speedup vs baseline: 1.2385x; 1.2354x over previous
import jax
import jax.numpy as jnp
from jax import lax
from jax.experimental import pallas as pl
from jax.experimental.pallas import tpu as pltpu
from jax.experimental.pallas import tpu_sc as plsc

NMAX = 48
D = 64
L = 3
PAIR = NMAX * NMAX
UB = 8
NU = NMAX // UB

N = 4096
E = 16384
B = 128
NS = 16
NC = 2
EPT = E // NS
NPT = N // NS
GPP = 4
SLAB = GPP * PAIR
ROWS_PT = SLAB // NS
PASSES = (B // NC) // GPP

_INTERPRET = False


_HB = N // 8
_EB = E // 8


def _enc_kernel(xi_ref, ei_ref, at_ref, bt_ref, h_ref, e_ref):
    f32 = jnp.float32
    ia = lax.broadcasted_iota(jnp.int32, (_HB, 9 * 64), 1)
    oh = jnp.zeros((_HB, 9 * 64), f32)
    for f in range(9):
        oh = oh + (ia == xi_ref[:, f][:, None]).astype(f32)
    h_ref[...] = jnp.maximum(
        jnp.dot(oh, at_ref[...], preferred_element_type=f32), 0.0)

    ib = lax.broadcasted_iota(jnp.int32, (_EB, 12), 1)
    ohe = jnp.zeros((_EB, 12), f32)
    for f in range(3):
        ohe = ohe + (ib == ei_ref[:, f][:, None]).astype(f32)
    e_ref[...] = jnp.maximum(
        jnp.dot(ohe, bt_ref[...], preferred_element_type=f32), 0.0)


def _tc_encode(xi2, ei2, at_flat, bt_flat):
    return pl.pallas_call(
        _enc_kernel,
        grid=(8,),
        in_specs=[
            pl.BlockSpec((_HB, 9), lambda i: (i, 0)),
            pl.BlockSpec((_EB, 3), lambda i: (i, 0)),
            pl.BlockSpec(at_flat.shape, lambda i: (0, 0)),
            pl.BlockSpec(bt_flat.shape, lambda i: (0, 0)),
        ],
        out_specs=[pl.BlockSpec((_HB, D), lambda i: (i, 0)),
                   pl.BlockSpec((_EB, D), lambda i: (i, 0))],
        out_shape=[jax.ShapeDtypeStruct((N, D), jnp.float32),
                   jax.ShapeDtypeStruct((E, D), jnp.float32)],
        compiler_params=pltpu.CompilerParams(
            dimension_semantics=("arbitrary",)),
        interpret=_INTERPRET,
    )(xi2, ei2, at_flat, bt_flat)


def _sc_body(e_hbm, h_hbm, fi_hbm, nfi_hbm, zz_hbm, a_out,
             ev_v, hv_v, fi_v, nfi_v, idx_v, idxn_v, slab, sem):
    c = lax.axis_index("c")
    s = lax.axis_index("s")

    descs = [
        pltpu.async_copy(e_hbm.at[pl.ds(s * EPT, EPT)], ev_v, sem),
        pltpu.async_copy(h_hbm.at[pl.ds(s * NPT, NPT)], hv_v, sem),
        pltpu.async_copy(fi_hbm.at[pl.ds(s * 8, 8)], fi_v, sem),
        pltpu.async_copy(nfi_hbm.at[pl.ds(s * 2, 2)], nfi_v, sem),
    ]
    for dd in descs:
        dd.wait()

    trash = jnp.int32(SLAB) + s
    for p in range(PASSES):
        base = (c * (B // NC) + p * GPP) * PAIR
        d0 = pltpu.async_copy(
            zz_hbm.at[pl.ds(s * ROWS_PT, ROWS_PT)],
            slab.at[pl.ds(s * ROWS_PT, ROWS_PT)], sem)
        d1 = pltpu.async_copy(
            zz_hbm.at[pl.ds(SLAB + s, 1)], slab.at[pl.ds(SLAB + s, 1)], sem)
        d0.wait()
        d1.wait()
        plsc.subcore_barrier()

        def eadj(j, carry):
            for k in range(8):
                t = fi_v[k, pl.ds(j * 16, 16)] - base
                ok = (t >= 0) & (t < SLAB)
                idx_v[k, pl.ds(j * 16, 16)] = jnp.where(ok, t, trash)
            return carry

        lax.fori_loop(0, 8, eadj, 0)

        def nadj(j, carry):
            for k in range(2):
                t = nfi_v[k, pl.ds(j * 16, 16)] - base
                ok = (t >= 0) & (t < SLAB)
                idxn_v[k, pl.ds(j * 16, 16)] = jnp.where(ok, t, trash)
            return carry

        lax.fori_loop(0, 8, nadj, 0)

        for k in range(8):
            def ecnt(j, acc, _k=k):
                grp = idx_v[_k, pl.ds(j * 16, 16)]
                return acc + jnp.where(grp != trash, 1, 0)

            tot = lax.fori_loop(
                0, 8, ecnt, jnp.zeros((16,), jnp.int32)).sum()

            @pl.when(tot > 0)
            def _(_k=k):
                pltpu.sync_copy(ev_v.at[pl.ds(_k * 128, 128)],
                                slab.at[idx_v.at[_k]], add=True)
        for k in range(2):
            def ncnt(j, acc, _k=k):
                grp = idxn_v[_k, pl.ds(j * 16, 16)]
                return acc + jnp.where(grp != trash, 1, 0)

            tot = lax.fori_loop(
                0, 8, ncnt, jnp.zeros((16,), jnp.int32)).sum()

            @pl.when(tot > 0)
            def _(_k=k):
                pltpu.sync_copy(hv_v.at[pl.ds(_k * 128, 128)],
                                slab.at[idxn_v.at[_k]], add=True)
        plsc.subcore_barrier()

        pltpu.sync_copy(slab.at[pl.ds(s * ROWS_PT, ROWS_PT)],
                        a_out.at[pl.ds(base + s * ROWS_PT, ROWS_PT)])
        plsc.subcore_barrier()


def _sc_scatter(e_rows, h_rows, fi2, nfi2, zz):
    f32 = jnp.float32
    i32 = jnp.int32
    mesh = plsc.VectorSubcoreMesh(core_axis_name="c", subcore_axis_name="s")
    fn = pl.kernel(
        _sc_body,
        out_type=jax.ShapeDtypeStruct((B * PAIR, D), f32),
        mesh=mesh,
        scratch_types=[
            pltpu.VMEM((EPT, D), f32),
            pltpu.VMEM((NPT, D), f32),
            pltpu.VMEM((8, 128), i32),
            pltpu.VMEM((2, 128), i32),
            pltpu.VMEM((8, 128), i32),
            pltpu.VMEM((2, 128), i32),
            pltpu.VMEM_SHARED((SLAB + NS, D), f32),
            pltpu.SemaphoreType.DMA,
        ],
        compiler_params=pltpu.CompilerParams(use_tc_tiling_on_sc=False,
                                             needs_layout_passes=False),
    )
    return fn(e_rows, h_rows, fi2, nfi2, zz)


def _tc_kernel(cnt_ref, a_ref, w1_ref, w2_ref, w3_ref,
               wout_ref, bout_ref, out_ref, h1t_scr, h2_scr, x_scr):
    b = pl.program_id(0)
    nv = jnp.minimum(cnt_ref[b], NMAX)
    f32 = jnp.float32

    r = lax.broadcasted_iota(jnp.int32, (PAIR, 1), 0)
    pmf = ((r // NMAX < nv) & (r % NMAX < nv)).astype(f32)
    cntp = (nv * nv).astype(f32) + 1e-6

    Z = a_ref[...]
    Zt = jnp.swapaxes(Z.reshape(NMAX, NMAX, D), 0, 1).reshape(PAIR, D)

    for l in range(L):
        h1t_scr[...] = jnp.maximum(
            jnp.dot(Zt, w1_ref[l], preferred_element_type=f32), 0.0)
        h2_scr[...] = jnp.maximum(
            jnp.dot(Z, w2_ref[l], preferred_element_type=f32), 0.0)
        zw3 = jnp.dot(Z, w3_ref[l], preferred_element_type=f32)
        x_scr[...] = zw3

        nblk = (nv + 7) // 8
        for ub in range(NU):
            def ein_body(wb, acc, _ub=ub):
                base = wb * (8 * NMAX)
                for j in range(8):
                    a = h1t_scr[pl.ds(base + j * NMAX + _ub * UB, UB), :]
                    bb = h2_scr[pl.ds(base + j * NMAX, NMAX), :]
                    acc = acc + a[:, None, :] * bb[None, :, :]
                return acc

            acc = lax.fori_loop(
                0, nblk, ein_body, jnp.zeros((UB, NMAX, D), f32))
            x_scr[pl.ds(ub * UB * NMAX, UB * NMAX), :] += acc.reshape(
                UB * NMAX, D)

        X = x_scr[...]
        mu = X.sum(axis=0) / cntp
        x2 = (X * X).sum(axis=0) / cntp
        var = x2 - mu * mu
        s = lax.rsqrt(var + 1e-5)
        Znew = jnp.maximum((X - mu[None, :]) * s[None, :], 0.0) * pmf
        Z = Znew
        if l < L - 1:
            Zt = jnp.swapaxes(
                Znew.reshape(NMAX, NMAX, D), 0, 1).reshape(PAIR, D)

    g = Z.sum(axis=0) / cntp
    val = (g * wout_ref[0, :]).sum() + bout_ref[0]
    out_ref[0, 0, :] = jnp.full((128,), val, dtype=jnp.float32)


def _dense_backend(A, cnt, W1, W2, W3, Wout, bout):
    out = pl.pallas_call(
        _tc_kernel,
        grid=(B,),
        in_specs=[
            pl.BlockSpec(memory_space=pltpu.SMEM),
            pl.BlockSpec((PAIR, D), lambda b: (b, 0)),
            pl.BlockSpec(W1.shape, lambda b: (0, 0, 0)),
            pl.BlockSpec(W2.shape, lambda b: (0, 0, 0)),
            pl.BlockSpec(W3.shape, lambda b: (0, 0, 0)),
            pl.BlockSpec((1, D), lambda b: (0, 0)),
            pl.BlockSpec(memory_space=pltpu.SMEM),
        ],
        out_specs=pl.BlockSpec((1, 1, 128), lambda b: (b, 0, 0)),
        out_shape=jax.ShapeDtypeStruct((B, 1, 128), jnp.float32),
        scratch_shapes=[pltpu.VMEM((PAIR, D), jnp.float32),
                        pltpu.VMEM((PAIR, D), jnp.float32),
                        pltpu.VMEM((PAIR, D), jnp.float32)],
        compiler_params=pltpu.CompilerParams(
            dimension_semantics=("arbitrary",)),
        interpret=_INTERPRET,
    )(cnt, A, W1, W2, W3, Wout.T, bout)
    return out[:, 0, :1]


def kernel(x, edge_index, edge_attr, batch0, atom_tables, bond_tables,
           W1, W2, W3, Wout, bout):
    i32 = jnp.int32

    counts = jnp.bincount(batch0, length=B)
    offsets = jnp.cumsum(counts) - counts
    local = jnp.arange(N, dtype=i32) - offsets[batch0].astype(i32)
    nvalid = local < NMAX
    lc = jnp.minimum(local, NMAX - 1)

    src, dst = edge_index[0], edge_index[1]
    gs = batch0[src]
    gd = batch0[dst]
    ls = lc[src]
    ld = lc[dst]
    ev = (gs == gd) & nvalid[src] & nvalid[dst]
    fi = jnp.where(ev, gs.astype(i32) * PAIR + ls * NMAX + ld, B * PAIR)
    nfi = jnp.where(nvalid, batch0.astype(i32) * PAIR + lc * (NMAX + 1),
                    B * PAIR)

    xi2 = x.astype(i32) + jnp.arange(9, dtype=i32)[None, :] * 64
    ei2 = edge_attr.astype(i32) + jnp.arange(3, dtype=i32)[None, :] * 4
    fi2 = fi.reshape(E // 128, 128)
    nfi2 = nfi.reshape(N // 128, 128)
    at_flat = atom_tables.reshape(9 * 64, D)
    bt_flat = bond_tables.reshape(3 * 4, D)

    h_rows, e_rows = _tc_encode(xi2, ei2, at_flat, bt_flat)

    zz = jnp.zeros((SLAB + NS, D), jnp.float32)
    A = _sc_scatter(e_rows, h_rows, fi2, nfi2, zz)

    return _dense_backend(A, counts.astype(i32), W1, W2, W3, Wout, bout)


# speedup vs baseline: 1.2527x; 1.0114x over previous
import jax
import jax.numpy as jnp
from jax import lax
from jax.experimental import pallas as pl
from jax.experimental.pallas import tpu as pltpu
from jax.experimental.pallas import tpu_sc as plsc

NMAX = 48
D = 64
L = 3
PAIR = NMAX * NMAX
UB = 8
NU = NMAX // UB

N = 4096
E = 16384
B = 128
NS = 16
NC = 2
EPT = E // NS
NPT = N // NS
GPP = 4
SLAB = GPP * PAIR
ROWS_PT = SLAB // NS
PASSES = (B // NC) // GPP

_INTERPRET = False


_HB = N // 8
_EB = E // 8


def _enc_kernel(xi_ref, ei_ref, at_ref, bt_ref, h_ref, e_ref):
    f32 = jnp.float32
    ia = lax.broadcasted_iota(jnp.int32, (_HB, 9 * 64), 1)
    oh = jnp.zeros((_HB, 9 * 64), f32)
    for f in range(9):
        oh = oh + (ia == xi_ref[:, f][:, None]).astype(f32)
    h_ref[...] = jnp.maximum(
        jnp.dot(oh, at_ref[...], preferred_element_type=f32), 0.0)

    ib = lax.broadcasted_iota(jnp.int32, (_EB, 12), 1)
    ohe = jnp.zeros((_EB, 12), f32)
    for f in range(3):
        ohe = ohe + (ib == ei_ref[:, f][:, None]).astype(f32)
    e_ref[...] = jnp.maximum(
        jnp.dot(ohe, bt_ref[...], preferred_element_type=f32), 0.0)


def _tc_encode(xi2, ei2, at_flat, bt_flat):
    return pl.pallas_call(
        _enc_kernel,
        grid=(8,),
        in_specs=[
            pl.BlockSpec((_HB, 9), lambda i: (i, 0)),
            pl.BlockSpec((_EB, 3), lambda i: (i, 0)),
            pl.BlockSpec(at_flat.shape, lambda i: (0, 0)),
            pl.BlockSpec(bt_flat.shape, lambda i: (0, 0)),
        ],
        out_specs=[pl.BlockSpec((_HB, D), lambda i: (i, 0)),
                   pl.BlockSpec((_EB, D), lambda i: (i, 0))],
        out_shape=[jax.ShapeDtypeStruct((N, D), jnp.float32),
                   jax.ShapeDtypeStruct((E, D), jnp.float32)],
        compiler_params=pltpu.CompilerParams(
            dimension_semantics=("arbitrary",)),
        interpret=_INTERPRET,
    )(xi2, ei2, at_flat, bt_flat)


def _sc_body(e_hbm, h_hbm, fi_hbm, nfi_hbm, zz_hbm, a_out,
             ev_v, hv_v, fi_v, nfi_v, idx_v, idxn_v, slab, sem):
    c = lax.axis_index("c")
    s = lax.axis_index("s")

    descs = [
        pltpu.async_copy(e_hbm.at[pl.ds(s * EPT, EPT)], ev_v, sem),
        pltpu.async_copy(h_hbm.at[pl.ds(s * NPT, NPT)], hv_v, sem),
        pltpu.async_copy(fi_hbm.at[pl.ds(s * 8, 8)], fi_v, sem),
        pltpu.async_copy(nfi_hbm.at[pl.ds(s * 2, 2)], nfi_v, sem),
    ]
    for dd in descs:
        dd.wait()

    trash = jnp.int32(SLAB) + s
    for p in range(PASSES):
        base = (c * (B // NC) + p * GPP) * PAIR
        d0 = pltpu.async_copy(
            zz_hbm.at[pl.ds(s * ROWS_PT, ROWS_PT)],
            slab.at[pl.ds(s * ROWS_PT, ROWS_PT)], sem)
        d1 = pltpu.async_copy(
            zz_hbm.at[pl.ds(SLAB + s, 1)], slab.at[pl.ds(SLAB + s, 1)], sem)
        d0.wait()
        d1.wait()
        plsc.subcore_barrier()

        def eadj(j, carry):
            for k in range(8):
                t = fi_v[k, pl.ds(j * 16, 16)] - base
                ok = (t >= 0) & (t < SLAB)
                idx_v[k, pl.ds(j * 16, 16)] = jnp.where(ok, t, trash)
            return carry

        lax.fori_loop(0, 8, eadj, 0)

        def nadj(j, carry):
            for k in range(2):
                t = nfi_v[k, pl.ds(j * 16, 16)] - base
                ok = (t >= 0) & (t < SLAB)
                idxn_v[k, pl.ds(j * 16, 16)] = jnp.where(ok, t, trash)
            return carry

        lax.fori_loop(0, 8, nadj, 0)

        for k in range(8):
            def ecnt(j, acc, _k=k):
                grp = idx_v[_k, pl.ds(j * 16, 16)]
                return acc + jnp.where(grp != trash, 1, 0)

            tot = lax.fori_loop(
                0, 8, ecnt, jnp.zeros((16,), jnp.int32)).sum()

            @pl.when(tot > 0)
            def _(_k=k):
                pltpu.sync_copy(ev_v.at[pl.ds(_k * 128, 128)],
                                slab.at[idx_v.at[_k]], add=True)
        for k in range(2):
            def ncnt(j, acc, _k=k):
                grp = idxn_v[_k, pl.ds(j * 16, 16)]
                return acc + jnp.where(grp != trash, 1, 0)

            tot = lax.fori_loop(
                0, 8, ncnt, jnp.zeros((16,), jnp.int32)).sum()

            @pl.when(tot > 0)
            def _(_k=k):
                pltpu.sync_copy(hv_v.at[pl.ds(_k * 128, 128)],
                                slab.at[idxn_v.at[_k]], add=True)
        plsc.subcore_barrier()

        pltpu.sync_copy(slab.at[pl.ds(s * ROWS_PT, ROWS_PT)],
                        a_out.at[pl.ds(base + s * ROWS_PT, ROWS_PT)])
        plsc.subcore_barrier()


def _sc_scatter(e_rows, h_rows, fi2, nfi2, zz):
    f32 = jnp.float32
    i32 = jnp.int32
    mesh = plsc.VectorSubcoreMesh(core_axis_name="c", subcore_axis_name="s")
    fn = pl.kernel(
        _sc_body,
        out_type=jax.ShapeDtypeStruct((B * PAIR, D), f32),
        mesh=mesh,
        scratch_types=[
            pltpu.VMEM((EPT, D), f32),
            pltpu.VMEM((NPT, D), f32),
            pltpu.VMEM((8, 128), i32),
            pltpu.VMEM((2, 128), i32),
            pltpu.VMEM((8, 128), i32),
            pltpu.VMEM((2, 128), i32),
            pltpu.VMEM_SHARED((SLAB + NS, D), f32),
            pltpu.SemaphoreType.DMA,
        ],
        compiler_params=pltpu.CompilerParams(use_tc_tiling_on_sc=False,
                                             needs_layout_passes=False),
    )
    return fn(e_rows, h_rows, fi2, nfi2, zz)


def _tc_kernel(cnt_ref, a_ref, w1_ref, w2_ref, w3_ref,
               wout_ref, bout_ref, out_ref, h1t_scr, h2_scr, x_scr):
    b = pl.program_id(0)
    nv = jnp.minimum(cnt_ref[b], NMAX)
    f32 = jnp.float32

    r = lax.broadcasted_iota(jnp.int32, (PAIR, 1), 0)
    pmf = ((r // NMAX < nv) & (r % NMAX < nv)).astype(f32)
    cntp = (nv * nv).astype(f32) + 1e-6

    Z = a_ref[...]
    Zt = jnp.swapaxes(Z.reshape(NMAX, NMAX, D), 0, 1).reshape(PAIR, D)

    for l in range(L):
        h1t_scr[...] = jnp.maximum(
            jnp.dot(Zt, w1_ref[l], preferred_element_type=f32), 0.0)
        h2_scr[...] = jnp.maximum(
            jnp.dot(Z, w2_ref[l], preferred_element_type=f32), 0.0)
        zw3 = jnp.dot(Z, w3_ref[l], preferred_element_type=f32)

        nblk = (nv + 7) // 8
        for ub in range(NU):
            def ein_body(wb, acc, _ub=ub):
                base = wb * (8 * NMAX)
                for j in range(8):
                    a = h1t_scr[pl.ds(base + j * NMAX + _ub * UB, UB), :]
                    bb = h2_scr[pl.ds(base + j * NMAX, NMAX), :]
                    acc = acc + a[:, None, :] * bb[None, :, :]
                return acc

            acc = lax.fori_loop(
                0, nblk, ein_body, jnp.zeros((UB, NMAX, D), f32))
            x_scr[pl.ds(ub * UB * NMAX, UB * NMAX), :] = (
                zw3[ub * UB * NMAX:(ub + 1) * UB * NMAX, :]
                + acc.reshape(UB * NMAX, D))

        X = x_scr[...]
        mu = X.sum(axis=0) / cntp
        x2 = (X * X).sum(axis=0) / cntp
        var = x2 - mu * mu
        s = lax.rsqrt(var + 1e-5)
        Znew = jnp.maximum((X - mu[None, :]) * s[None, :], 0.0) * pmf
        Z = Znew
        if l < L - 1:
            Zt = jnp.swapaxes(
                Znew.reshape(NMAX, NMAX, D), 0, 1).reshape(PAIR, D)

    g = Z.sum(axis=0) / cntp
    val = (g * wout_ref[0, :]).sum() + bout_ref[0]
    out_ref[0, 0, :] = jnp.full((128,), val, dtype=jnp.float32)


def _dense_backend(A, cnt, W1, W2, W3, Wout, bout):
    out = pl.pallas_call(
        _tc_kernel,
        grid=(B,),
        in_specs=[
            pl.BlockSpec(memory_space=pltpu.SMEM),
            pl.BlockSpec((PAIR, D), lambda b: (b, 0)),
            pl.BlockSpec(W1.shape, lambda b: (0, 0, 0)),
            pl.BlockSpec(W2.shape, lambda b: (0, 0, 0)),
            pl.BlockSpec(W3.shape, lambda b: (0, 0, 0)),
            pl.BlockSpec((1, D), lambda b: (0, 0)),
            pl.BlockSpec(memory_space=pltpu.SMEM),
        ],
        out_specs=pl.BlockSpec((1, 1, 128), lambda b: (b, 0, 0)),
        out_shape=jax.ShapeDtypeStruct((B, 1, 128), jnp.float32),
        scratch_shapes=[pltpu.VMEM((PAIR, D), jnp.float32),
                        pltpu.VMEM((PAIR, D), jnp.float32),
                        pltpu.VMEM((PAIR, D), jnp.float32)],
        compiler_params=pltpu.CompilerParams(
            dimension_semantics=("arbitrary",)),
        interpret=_INTERPRET,
    )(cnt, A, W1, W2, W3, Wout.T, bout)
    return out[:, 0, :1]


def kernel(x, edge_index, edge_attr, batch0, atom_tables, bond_tables,
           W1, W2, W3, Wout, bout):
    i32 = jnp.int32

    counts = jnp.bincount(batch0, length=B)
    offsets = jnp.cumsum(counts) - counts
    local = jnp.arange(N, dtype=i32) - offsets[batch0].astype(i32)
    nvalid = local < NMAX
    lc = jnp.minimum(local, NMAX - 1)

    src, dst = edge_index[0], edge_index[1]
    gs = batch0[src]
    gd = batch0[dst]
    ls = lc[src]
    ld = lc[dst]
    ev = (gs == gd) & nvalid[src] & nvalid[dst]
    fi = jnp.where(ev, gs.astype(i32) * PAIR + ls * NMAX + ld, B * PAIR)
    nfi = jnp.where(nvalid, batch0.astype(i32) * PAIR + lc * (NMAX + 1),
                    B * PAIR)

    xi2 = x.astype(i32) + jnp.arange(9, dtype=i32)[None, :] * 64
    ei2 = edge_attr.astype(i32) + jnp.arange(3, dtype=i32)[None, :] * 4
    fi2 = fi.reshape(E // 128, 128)
    nfi2 = nfi.reshape(N // 128, 128)
    at_flat = atom_tables.reshape(9 * 64, D)
    bt_flat = bond_tables.reshape(3 * 4, D)

    h_rows, e_rows = _tc_encode(xi2, ei2, at_flat, bt_flat)

    zz = jnp.zeros((SLAB + NS, D), jnp.float32)
    A = _sc_scatter(e_rows, h_rows, fi2, nfi2, zz)

    return _dense_backend(A, counts.astype(i32), W1, W2, W3, Wout, bout)
